# Initial kernel scaffold; baseline (speedup 1.0000x reference)
import jax
import jax.numpy as jnp
from jax.experimental import pallas as pl


def kernel(u, edge_index, neighbor_all, emb_id, att_fc1_w, att_fc1_b, att_fc2_w, att_fc2_b, w, fc1_w, fc1_b, fc2_w, fc2_b):
    raise NotImplementedError("write your pallas kernel here")


# speedup vs baseline: 1.9811x; 1.9811x over previous
import functools

import jax
import jax.numpy as jnp
from jax import lax
from jax.experimental import pallas as pl
from jax.experimental.pallas import tpu as pltpu
from jax.experimental.pallas import tpu_sc as plsc

H = 128
NW = 32
LANES = 16


def _edge_diffprod_kernel(E, EW, CH):
    n_chunks = EW // CH

    mesh = plsc.VectorSubcoreMesh(core_axis_name="c", subcore_axis_name="s")

    @functools.partial(
        pl.kernel,
        out_type=jax.ShapeDtypeStruct((E, H), jnp.float32),
        mesh=mesh,
        scratch_types=[
            pltpu.VMEM((CH,), jnp.int32),
            pltpu.VMEM((CH,), jnp.int32),
            pltpu.VMEM((CH, 2 * H), jnp.float32),
            pltpu.VMEM((CH, 2 * H), jnp.float32),
            pltpu.VMEM((CH, H), jnp.float32),
            pltpu.SemaphoreType.DMA,
            pltpu.SemaphoreType.DMA,
        ],
    )
    def edge_kernel(t_hbm, k_hbm, i_hbm, h_hbm, kidx, iidx, krows, irows,
                    hbuf, sem1, sem2):
        wid = lax.axis_index("s") * 2 + lax.axis_index("c")
        base = wid * EW

        def chunk_body(c, carry):
            off = base + c * CH
            pltpu.sync_copy(k_hbm.at[pl.ds(off, CH)], kidx)
            pltpu.sync_copy(i_hbm.at[pl.ds(off, CH)], iidx)
            cp1 = pltpu.async_copy(t_hbm.at[kidx], krows, sem1)
            cp2 = pltpu.async_copy(t_hbm.at[iidx], irows, sem2)
            cp1.wait()
            cp2.wait()

            def row_body(e, carry2):
                for l in range(H // LANES):
                    o = l * LANES
                    du = krows[e, pl.ds(o, LANES)] - irows[e, pl.ds(o, LANES)]
                    dx = (krows[e, pl.ds(H + o, LANES)]
                          - irows[e, pl.ds(H + o, LANES)])
                    hbuf[e, pl.ds(o, LANES)] = du * dx
                return carry2

            lax.fori_loop(0, CH, row_body, 0)
            pltpu.sync_copy(hbuf, h_hbm.at[pl.ds(off, CH)])
            return carry

        lax.fori_loop(0, n_chunks, chunk_body, 0)

    return edge_kernel


def _logits_call(h0, w1t, b1, w2, E, EB):
    def body(h_ref, w1t_ref, b1_ref, w2_ref, out_ref):
        h = jnp.dot(h_ref[...], w1t_ref[...],
                    preferred_element_type=jnp.float32)
        h = jnp.maximum(h + b1_ref[...], 0.0)
        out_ref[...] = lax.dot_general(
            w2_ref[...], h, (((1,), (1,)), ((), ())),
            preferred_element_type=jnp.float32).reshape(1, 1, EB)

    return pl.pallas_call(
        body,
        grid=(E // EB,),
        in_specs=[
            pl.BlockSpec((EB, H), lambda b: (b, 0)),
            pl.BlockSpec((H, H), lambda b: (0, 0)),
            pl.BlockSpec((1, H), lambda b: (0, 0)),
            pl.BlockSpec((1, H), lambda b: (0, 0)),
        ],
        out_specs=pl.BlockSpec((1, 1, EB), lambda b: (b, 0, 0)),
        out_shape=jax.ShapeDtypeStruct((E // EB, 1, EB), jnp.float32),
    )(h0, w1t, b1, w2)


def _softmax_call(logits2d):
    def body(l_ref, out_ref):
        l = l_ref[...]
        m = jnp.max(l)
        e = jnp.exp(l - m)
        out_ref[...] = e / jnp.sum(e)

    return pl.pallas_call(
        body,
        out_shape=jax.ShapeDtypeStruct(logits2d.shape, jnp.float32),
    )(logits2d)


def _neighbor_agg_kernel(N, D, E_pad, CN):
    PAIRS = CN * D
    total_chunks = N // CN
    base_chunks = total_chunks // NW
    extra = total_chunks - base_chunks * NW

    mesh = plsc.VectorSubcoreMesh(core_axis_name="c", subcore_axis_name="s")

    @functools.partial(
        pl.kernel,
        out_type=jax.ShapeDtypeStruct((N, H), jnp.float32),
        mesh=mesh,
        scratch_types=[
            pltpu.VMEM((PAIRS,), jnp.int32),
            pltpu.VMEM((PAIRS,), jnp.int32),
            pltpu.VMEM((PAIRS + LANES,), jnp.float32),
            pltpu.VMEM((PAIRS, H), jnp.float32),
            pltpu.VMEM((CN, H), jnp.float32),
            pltpu.SemaphoreType.DMA,
            pltpu.SemaphoreType.DMA,
            pltpu.SemaphoreType.DMA,
        ],
    )
    def agg_kernel(naf_hbm, kpad_hbm, apad_hbm, x_hbm, agg_hbm,
                   jidx, kj, av, rows, outbuf, sem1, sem2, sem3):
        wid = lax.axis_index("s") * 2 + lax.axis_index("c")
        my_chunks = base_chunks + jnp.where(wid < extra, 1, 0)
        start = base_chunks * wid + jnp.minimum(wid, extra)

        def chunk_body(c, carry):
            g0 = (start + c) * CN
            pltpu.sync_copy(naf_hbm.at[pl.ds(g0 * D, PAIRS)], jidx)
            cpk = pltpu.async_copy(kpad_hbm.at[jidx], kj, sem1)
            cpa = pltpu.async_copy(apad_hbm.at[jidx], av.at[pl.ds(0, PAIRS)],
                                   sem2)
            cpk.wait()
            cpa.wait()
            cpr = pltpu.async_copy(x_hbm.at[kj], rows, sem3)
            cpr.wait()

            for n in range(CN):
                def d_body(d, acc):
                    cidx = n * D + d
                    a = av[pl.ds(cidx, LANES)][0]
                    return tuple(
                        acc[l] + a * rows[cidx, pl.ds(l * LANES, LANES)]
                        for l in range(H // LANES))

                zero = jnp.zeros((LANES,), jnp.float32)
                acc = lax.fori_loop(0, D, d_body,
                                    tuple(zero for _ in range(H // LANES)))
                for l in range(H // LANES):
                    outbuf[n, pl.ds(l * LANES, LANES)] = acc[l]

            pltpu.sync_copy(outbuf, agg_hbm.at[pl.ds(g0, CN)])
            return carry

        lax.fori_loop(0, my_chunks, chunk_body, 0)

    return agg_kernel


def _head_call(x, agg, w0, w1, fc1t, fc1_b, fc2t, fc2_b, N, NB, OUT):
    def body(x_ref, agg_ref, w0_ref, w1_ref, fc1t_ref, fc1b_ref, fc2t_ref,
             fc2b_ref, out_ref):
        x2 = (jnp.dot(x_ref[...], w0_ref[...],
                      preferred_element_type=jnp.float32)
              + jnp.dot(agg_ref[...], w1_ref[...],
                        preferred_element_type=jnp.float32))
        x2 = jnp.maximum(jnp.dot(x2, fc1t_ref[...],
                                 preferred_element_type=jnp.float32)
                         + fc1b_ref[...], 0.0)
        out_ref[...] = jnp.dot(x2, fc2t_ref[...],
                               preferred_element_type=jnp.float32) \
            + fc2b_ref[...]

    return pl.pallas_call(
        body,
        grid=(N // NB,),
        in_specs=[
            pl.BlockSpec((NB, H), lambda b: (b, 0)),
            pl.BlockSpec((NB, H), lambda b: (b, 0)),
            pl.BlockSpec((H, H), lambda b: (0, 0)),
            pl.BlockSpec((H, H), lambda b: (0, 0)),
            pl.BlockSpec((H, H), lambda b: (0, 0)),
            pl.BlockSpec((1, H), lambda b: (0, 0)),
            pl.BlockSpec((H, OUT), lambda b: (0, 0)),
            pl.BlockSpec((1, OUT), lambda b: (0, 0)),
        ],
        out_specs=pl.BlockSpec((NB, OUT), lambda b: (b, 0)),
        out_shape=jax.ShapeDtypeStruct((N, OUT), jnp.float32),
    )(x, agg, w0, w1, fc1t, fc1_b, fc2t, fc2_b)


def kernel(u, edge_index, neighbor_all, emb_id, att_fc1_w, att_fc1_b,
           att_fc2_w, att_fc2_b, w, fc1_w, fc1_b, fc2_w, fc2_b):
    N, Hdim = u.shape
    E = edge_index.shape[1]
    D = neighbor_all.shape[1]
    OUT = fc2_w.shape[0]
    assert Hdim == H

    x = emb_id
    k = edge_index[0]
    i = edge_index[1]

    t = jnp.concatenate([u, x], axis=1)
    EW = E // NW
    CH = 80
    h0 = _edge_diffprod_kernel(E, EW, CH)(t, k, i)

    EB = 512
    logits = _logits_call(h0, att_fc1_w.T, att_fc1_b.reshape(1, H),
                          att_fc2_w, E, EB).reshape(E // EB, EB)
    alpha2d = _softmax_call(logits)

    PAD = 8
    alpha_pad = jnp.concatenate(
        [alpha2d.reshape(E), jnp.zeros((PAD,), jnp.float32)])
    k_pad = jnp.concatenate([k, jnp.zeros((PAD,), jnp.int32)])
    naf = neighbor_all.reshape(N * D)
    CN = 128 // D
    agg = _neighbor_agg_kernel(N, D, E + PAD, CN)(naf, k_pad, alpha_pad, x)

    NB = 1000
    return _head_call(x, agg, w[0], w[1], fc1_w.T, fc1_b.reshape(1, H),
                      fc2_w.T, fc2_b.reshape(1, OUT), N, NB, OUT)


# speedup vs baseline: 2.9133x; 1.4705x over previous
import functools

import jax
import jax.numpy as jnp
from jax import lax
from jax.experimental import pallas as pl
from jax.experimental.pallas import tpu as pltpu
from jax.experimental.pallas import tpu_sc as plsc

H = 128
NW = 32
LANES = 16


def _edge_diffprod_kernel(E, EW, CH):
    n_chunks = EW // CH
    n_pairs = n_chunks // 2
    assert n_chunks == 2 * n_pairs + 1

    mesh = plsc.VectorSubcoreMesh(core_axis_name="c", subcore_axis_name="s")

    @functools.partial(
        pl.kernel,
        out_type=jax.ShapeDtypeStruct((E, H), jnp.float32),
        mesh=mesh,
        scratch_types=[
            pltpu.VMEM((EW,), jnp.int32),
            pltpu.VMEM((EW,), jnp.int32),
            pltpu.VMEM((2, CH, 2 * H), jnp.float32),
            pltpu.VMEM((2, CH, 2 * H), jnp.float32),
            pltpu.VMEM((2, CH, H), jnp.float32),
            pltpu.SemaphoreType.DMA,
            pltpu.SemaphoreType.DMA,
            pltpu.SemaphoreType.DMA,
            pltpu.SemaphoreType.DMA,
            pltpu.SemaphoreType.DMA,
            pltpu.SemaphoreType.DMA,
        ],
    )
    def edge_kernel(t_hbm, k_hbm, i_hbm, h_hbm, kidx_all, iidx_all,
                    krows, irows, hbuf, sk0, sk1, si0, si1, st0, st1):
        wid = lax.axis_index("s") * 2 + lax.axis_index("c")
        base = wid * EW
        pltpu.sync_copy(k_hbm.at[pl.ds(base, EW)], kidx_all)
        pltpu.sync_copy(i_hbm.at[pl.ds(base, EW)], iidx_all)
        semk = (sk0, sk1)
        semi = (si0, si1)
        semst = (st0, st1)

        def fire(c, b):
            pltpu.async_copy(
                t_hbm.at[kidx_all.at[pl.ds(c * CH, CH)]], krows.at[b],
                semk[b])
            pltpu.async_copy(
                t_hbm.at[iidx_all.at[pl.ds(c * CH, CH)]], irows.at[b],
                semi[b])

        def wait_gather(b):
            pltpu.make_async_copy(
                t_hbm.at[kidx_all.at[pl.ds(0, CH)]], krows.at[b],
                semk[b]).wait()
            pltpu.make_async_copy(
                t_hbm.at[iidx_all.at[pl.ds(0, CH)]], irows.at[b],
                semi[b]).wait()

        def compute(b):
            def row_body(e, carry2):
                for l in range(H // LANES):
                    o = l * LANES
                    du = (krows[b, e, pl.ds(o, LANES)]
                          - irows[b, e, pl.ds(o, LANES)])
                    dx = (krows[b, e, pl.ds(H + o, LANES)]
                          - irows[b, e, pl.ds(H + o, LANES)])
                    hbuf[b, e, pl.ds(o, LANES)] = du * dx
                return carry2

            lax.fori_loop(0, CH, row_body, 0)

        def fire_store(c, b):
            pltpu.async_copy(hbuf.at[b],
                             h_hbm.at[pl.ds(base + c * CH, CH)], semst[b])

        def wait_store(b):
            pltpu.make_async_copy(hbuf.at[b], h_hbm.at[pl.ds(base, CH)],
                                  semst[b]).wait()

        fire(0, 0)
        fire(1, 1)

        def pair_body(p, carry):
            c0 = 2 * p
            wait_gather(0)

            @pl.when(p > 0)
            def _():
                wait_store(0)

            compute(0)
            fire_store(c0, 0)
            fire(c0 + 2, 0)
            wait_gather(1)

            @pl.when(p > 0)
            def _():
                wait_store(1)

            compute(1)
            fire_store(c0 + 1, 1)

            @pl.when(p < n_pairs - 1)
            def _():
                fire(c0 + 3, 1)

            return carry

        lax.fori_loop(0, n_pairs, pair_body, 0)

        wait_gather(0)
        wait_store(0)
        compute(0)
        pltpu.sync_copy(hbuf.at[0],
                        h_hbm.at[pl.ds(base + (n_chunks - 1) * CH, CH)])
        wait_store(1)

    return edge_kernel


def _logits_call(h0, w1t, b1, w2, E, EB):
    def body(h_ref, w1t_ref, b1_ref, w2_ref, out_ref):
        h = jnp.dot(h_ref[...], w1t_ref[...],
                    preferred_element_type=jnp.float32)
        h = jnp.maximum(h + b1_ref[...], 0.0)
        out_ref[...] = lax.dot_general(
            w2_ref[...], h, (((1,), (1,)), ((), ())),
            preferred_element_type=jnp.float32).reshape(1, 1, EB)

    return pl.pallas_call(
        body,
        grid=(E // EB,),
        in_specs=[
            pl.BlockSpec((EB, H), lambda b: (b, 0)),
            pl.BlockSpec((H, H), lambda b: (0, 0)),
            pl.BlockSpec((1, H), lambda b: (0, 0)),
            pl.BlockSpec((1, H), lambda b: (0, 0)),
        ],
        out_specs=pl.BlockSpec((1, 1, EB), lambda b: (b, 0, 0)),
        out_shape=jax.ShapeDtypeStruct((E // EB, 1, EB), jnp.float32),
    )(h0, w1t, b1, w2)


def _softmax_call(logits2d):
    def body(l_ref, out_ref):
        l = l_ref[...]
        m = jnp.max(l)
        e = jnp.exp(l - m)
        out_ref[...] = e / jnp.sum(e)

    return pl.pallas_call(
        body,
        out_shape=jax.ShapeDtypeStruct(logits2d.shape, jnp.float32),
    )(logits2d)


def _neighbor_agg_kernel(N, D, CN):
    PAIRS = CN * D
    nodes_w = (N // NW) // CN * CN
    n_chunks = nodes_w // CN
    n_pairs = n_chunks // 2
    assert n_chunks == 2 * n_pairs
    rem = N - NW * nodes_w
    assert rem <= NW
    jpre = n_chunks * PAIRS

    mesh = plsc.VectorSubcoreMesh(core_axis_name="c", subcore_axis_name="s")

    @functools.partial(
        pl.kernel,
        out_type=jax.ShapeDtypeStruct((N, H), jnp.float32),
        mesh=mesh,
        scratch_types=[
            pltpu.VMEM((jpre,), jnp.int32),
            pltpu.VMEM((2, PAIRS), jnp.int32),
            pltpu.VMEM((PAIRS + LANES,), jnp.float32),
            pltpu.VMEM((PAIRS + LANES,), jnp.float32),
            pltpu.VMEM((2, PAIRS, H), jnp.float32),
            pltpu.VMEM((nodes_w, H), jnp.float32),
            pltpu.VMEM((D,), jnp.int32),
            pltpu.VMEM((D,), jnp.int32),
            pltpu.VMEM((D + LANES,), jnp.float32),
            pltpu.VMEM((D, H), jnp.float32),
            pltpu.SemaphoreType.DMA,
            pltpu.SemaphoreType.DMA,
            pltpu.SemaphoreType.DMA,
            pltpu.SemaphoreType.DMA,
            pltpu.SemaphoreType.DMA,
            pltpu.SemaphoreType.DMA,
        ],
    )
    def agg_kernel(naf_hbm, kpad_hbm, apad_hbm, x_hbm, agg_hbm,
                   jidx_all, kj, av0, av1, rows, outall, ej, ekj, eav, erows,
                   ska, skb, saa, sab, sra, srb):
        av = (av0, av1)
        wid = lax.axis_index("s") * 2 + lax.axis_index("c")
        node0 = wid * nodes_w
        pltpu.sync_copy(naf_hbm.at[pl.ds(node0 * D, jpre)], jidx_all)
        semk = (ska, skb)
        sema = (saa, sab)
        semr = (sra, srb)

        def fire_kjav(c, b):
            idx = jidx_all.at[pl.ds(c * PAIRS, PAIRS)]
            pltpu.async_copy(kpad_hbm.at[idx], kj.at[b], semk[b])
            pltpu.async_copy(apad_hbm.at[idx],
                             av[b].at[pl.ds(0, PAIRS)], sema[b])

        def wait_kjav(b):
            idx = jidx_all.at[pl.ds(0, PAIRS)]
            pltpu.make_async_copy(kpad_hbm.at[idx], kj.at[b], semk[b]).wait()
            pltpu.make_async_copy(apad_hbm.at[idx],
                                  av[b].at[pl.ds(0, PAIRS)],
                                  sema[b]).wait()

        def fire_rows(b):
            pltpu.async_copy(x_hbm.at[kj.at[b]], rows.at[b], semr[b])

        def wait_rows(b):
            pltpu.make_async_copy(x_hbm.at[kj.at[b]], rows.at[b],
                                  semr[b]).wait()

        def compute(c, b):
            for n in range(CN):
                def d_body(d, acc):
                    cidx = n * D + d
                    a = av[b][pl.ds(cidx, LANES)][0]
                    return tuple(
                        acc[l] + a * rows[b, cidx, pl.ds(l * LANES, LANES)]
                        for l in range(H // LANES))

                zero = jnp.zeros((LANES,), jnp.float32)
                acc = lax.fori_loop(0, D, d_body,
                                    tuple(zero for _ in range(H // LANES)))
                row = c * CN + n
                for l in range(H // LANES):
                    outall[row, pl.ds(l * LANES, LANES)] = acc[l]

        fire_kjav(0, 0)
        wait_kjav(0)
        fire_rows(0)
        fire_kjav(1, 1)

        def pair_body(p, carry):
            c0 = 2 * p
            wait_kjav(1)
            fire_rows(1)
            wait_rows(0)
            compute(c0, 0)

            @pl.when(p < n_pairs - 1)
            def _():
                fire_kjav(c0 + 2, 0)

            wait_rows(1)
            compute(c0 + 1, 1)

            @pl.when(p < n_pairs - 1)
            def _():
                wait_kjav(0)
                fire_rows(0)
                fire_kjav(c0 + 3, 1)

            return carry

        lax.fori_loop(0, n_pairs, pair_body, 0)
        pltpu.sync_copy(outall, agg_hbm.at[pl.ds(node0, nodes_w)])

        @pl.when(wid < rem)
        def _():
            g = NW * nodes_w + wid
            pltpu.sync_copy(naf_hbm.at[pl.ds(g * D, D)], ej)
            cpk = pltpu.async_copy(kpad_hbm.at[ej], ekj, ska)
            cpa = pltpu.async_copy(apad_hbm.at[ej], eav.at[pl.ds(0, D)], saa)
            cpk.wait()
            cpa.wait()
            cpr = pltpu.async_copy(x_hbm.at[ekj], erows, sra)
            cpr.wait()

            def d_body(d, acc):
                a = eav[pl.ds(d, LANES)][0]
                return tuple(
                    acc[l] + a * erows[d, pl.ds(l * LANES, LANES)]
                    for l in range(H // LANES))

            zero = jnp.zeros((LANES,), jnp.float32)
            acc = lax.fori_loop(0, D, d_body,
                                tuple(zero for _ in range(H // LANES)))
            for l in range(H // LANES):
                erows[0, pl.ds(l * LANES, LANES)] = acc[l]
            pltpu.sync_copy(erows.at[pl.ds(0, 1)], agg_hbm.at[pl.ds(g, 1)])

    return agg_kernel


def _head_call(x, agg, w0, w1, fc1t, fc1_b, fc2t, fc2_b, N, NB, OUT):
    def body(x_ref, agg_ref, w0_ref, w1_ref, fc1t_ref, fc1b_ref, fc2t_ref,
             fc2b_ref, out_ref):
        x2 = (jnp.dot(x_ref[...], w0_ref[...],
                      preferred_element_type=jnp.float32)
              + jnp.dot(agg_ref[...], w1_ref[...],
                        preferred_element_type=jnp.float32))
        x2 = jnp.maximum(jnp.dot(x2, fc1t_ref[...],
                                 preferred_element_type=jnp.float32)
                         + fc1b_ref[...], 0.0)
        out_ref[...] = jnp.dot(x2, fc2t_ref[...],
                               preferred_element_type=jnp.float32) \
            + fc2b_ref[...]

    return pl.pallas_call(
        body,
        grid=(N // NB,),
        in_specs=[
            pl.BlockSpec((NB, H), lambda b: (b, 0)),
            pl.BlockSpec((NB, H), lambda b: (b, 0)),
            pl.BlockSpec((H, H), lambda b: (0, 0)),
            pl.BlockSpec((H, H), lambda b: (0, 0)),
            pl.BlockSpec((H, H), lambda b: (0, 0)),
            pl.BlockSpec((1, H), lambda b: (0, 0)),
            pl.BlockSpec((H, OUT), lambda b: (0, 0)),
            pl.BlockSpec((1, OUT), lambda b: (0, 0)),
        ],
        out_specs=pl.BlockSpec((NB, OUT), lambda b: (b, 0)),
        out_shape=jax.ShapeDtypeStruct((N, OUT), jnp.float32),
    )(x, agg, w0, w1, fc1t, fc1_b, fc2t, fc2_b)


def kernel(u, edge_index, neighbor_all, emb_id, att_fc1_w, att_fc1_b,
           att_fc2_w, att_fc2_b, w, fc1_w, fc1_b, fc2_w, fc2_b):
    N, Hdim = u.shape
    E = edge_index.shape[1]
    D = neighbor_all.shape[1]
    OUT = fc2_w.shape[0]
    assert Hdim == H

    x = emb_id
    k = edge_index[0]
    i = edge_index[1]

    t = jnp.concatenate([u, x], axis=1)
    EW = E // NW
    CH = 80
    h0 = _edge_diffprod_kernel(E, EW, CH)(t, k, i)

    EB = 512
    logits = _logits_call(h0, att_fc1_w.T, att_fc1_b.reshape(1, H),
                          att_fc2_w, E, EB).reshape(E // EB, EB)
    alpha2d = _softmax_call(logits)

    PAD = 8
    alpha_pad = jnp.concatenate(
        [alpha2d.reshape(E), jnp.zeros((PAD,), jnp.float32)])
    k_pad = jnp.concatenate([k, jnp.zeros((PAD,), jnp.int32)])
    naf = neighbor_all.reshape(N * D)
    CN = 128 // D
    agg = _neighbor_agg_kernel(N, D, CN)(naf, k_pad, alpha_pad, x)

    NB = 1000
    return _head_call(x, agg, w[0], w[1], fc1_w.T, fc1_b.reshape(1, H),
                      fc2_w.T, fc2_b.reshape(1, OUT), N, NB, OUT)
